# baseline (speedup 1.0000x reference)
import functools

import jax
import jax.numpy as jnp
from jax import lax
from jax.experimental import pallas as pl
from jax.experimental.pallas import tpu as pltpu
from jax.experimental.pallas import tpu_sc as plsc

NC = 2
NS = 16
NW = NC * NS
CHUNK = 128
PACK = 4
OUTC = CHUNK // PACK


@functools.partial(jax.jit, static_argnames=("num_chunks",))
def _sc_gather(pidx, qoff, table2, *, num_chunks):
    total = NW * num_chunks * CHUNK

    def body(pidx_hbm, qoff_hbm, table_hbm, out_hbm, pidx_v, qoff_v,
             g0, g1, o0, o1, gsem0, gsem1, ssem0, ssem1):
        wid = lax.axis_index("s") * NC + lax.axis_index("c")
        pltpu.sync_copy(pidx_hbm.at[wid], pidx_v)
        pltpu.sync_copy(qoff_hbm.at[wid], qoff_v)
        base = wid * (num_chunks * OUTC)
        G = (g0, g1)
        O = (o0, o1)
        gsem = (gsem0, gsem1)
        ssem = (ssem0, ssem1)

        def fire_gather(c, p):
            pltpu.async_copy(table_hbm.at[pidx_v.at[c, 0]], G[p], gsem[p])

        def wait_gather(p):
            pltpu.make_async_copy(
                table_hbm.at[pidx_v.at[0, 0]], G[p], gsem[p]).wait()

        def fire_store(c, p):
            pltpu.async_copy(
                O[p], out_hbm.at[pl.ds(base + c * OUTC, OUTC)], ssem[p])

        def wait_store(p):
            pltpu.make_async_copy(
                O[p], out_hbm.at[pl.ds(0, OUTC)], ssem[p]).wait()

        lanes = lax.iota(jnp.int32, 16)

        def extract(c, p):
            for jb in range(OUTC // 16):
                jvec = jb * 16 + lanes
                for k in range(PACK):
                    rowids = PACK * jvec + k
                    q = plsc.load_gather(qoff_v.at[c, 0], [rowids])
                    for c2 in range(32):
                        val = plsc.load_gather(G[p], [rowids, q + c2])
                        plsc.store_scatter(
                            O[p], [jvec, jnp.full((16,), 32 * k + c2,
                                                  jnp.int32)], val)

        def two_chunks(t, carry):
            for h in range(2):
                c = 2 * t + h
                p = h
                q2 = 1 - h
                @pl.when(c + 1 < num_chunks)
                def _():
                    @pl.when(c >= 1)
                    def _():
                        wait_store(q2)
                    fire_gather(c + 1, q2)
                wait_gather(p)
                extract(c, p)
                fire_store(c, p)
            return carry

        fire_gather(0, 0)
        lax.fori_loop(0, num_chunks // 2, two_chunks, 0, unroll=False)
        wait_store(0)
        wait_store(1)

    grid_kernel = pl.kernel(
        body,
        out_type=jax.ShapeDtypeStruct((total // PACK, 128), jnp.float32),
        mesh=plsc.VectorSubcoreMesh(
            core_axis_name="c", subcore_axis_name="s", num_cores=NC,
            num_subcores=NS),
        scratch_types=[
            pltpu.VMEM((num_chunks, 1, CHUNK), jnp.int32),
            pltpu.VMEM((num_chunks, 1, CHUNK), jnp.int32),
            pltpu.VMEM((CHUNK, 128), jnp.float32),
            pltpu.VMEM((CHUNK, 128), jnp.float32),
            pltpu.VMEM((OUTC, 128), jnp.float32),
            pltpu.VMEM((OUTC, 128), jnp.float32),
            pltpu.SemaphoreType.DMA,
            pltpu.SemaphoreType.DMA,
            pltpu.SemaphoreType.DMA,
            pltpu.SemaphoreType.DMA,
        ],
        compiler_params=pltpu.CompilerParams(
            use_tc_tiling_on_sc=True, needs_layout_passes=False),
    )
    return grid_kernel(pidx, qoff, table2)


def kernel(indices, table):
    batch, hist = indices.shape
    dim = table.shape[1]
    total = batch * hist
    assert total % (NW * CHUNK) == 0
    num_chunks = total // (NW * CHUNK)
    assert num_chunks % 2 == 0
    flat = indices.reshape(-1).astype(jnp.int32)
    pidx = (flat // PACK).reshape(NW, num_chunks, 1, CHUNK)
    qoff = ((flat % PACK) * dim).reshape(NW, num_chunks, 1, CHUNK)
    table2 = table.reshape(table.shape[0] // PACK, PACK * dim)
    out = _sc_gather(pidx, qoff, table2, num_chunks=num_chunks)
    return out.reshape(batch, hist, dim)


# speedup vs baseline: 1.3845x; 1.3845x over previous
import functools

import jax
import jax.numpy as jnp
from jax import lax
from jax.experimental import pallas as pl
from jax.experimental.pallas import tpu as pltpu
from jax.experimental.pallas import tpu_sc as plsc

NC = 2
NS = 16
NW = NC * NS
CHUNK = 800


@functools.partial(jax.jit, static_argnames=("num_chunks", "dim", "hist"))
def _sc_gather(idx, table, *, num_chunks, dim, hist):
    total = NW * num_chunks * CHUNK
    batches = total // hist
    bpc = CHUNK // hist

    def body(idx_hbm, table_hbm, out_hbm, idx_v, rows0, rows1, gsem0, gsem1,
             ssem0, ssem1):
        wid = lax.axis_index("s") * NC + lax.axis_index("c")
        pltpu.sync_copy(idx_hbm.at[wid], idx_v)
        base = wid * (num_chunks * bpc)
        rows = (rows0, rows1)
        gsem = (gsem0, gsem1)
        ssem = (ssem0, ssem1)

        def fire_gather(c, p):
            pltpu.async_copy(table_hbm.at[idx_v.at[c, 0]], rows[p], gsem[p])

        def wait_gather(p):
            pltpu.make_async_copy(
                table_hbm.at[idx_v.at[0, 0]], rows[p], gsem[p]).wait()

        def fire_store(c, p):
            for b in range(bpc):
                pltpu.async_copy(
                    rows[p].at[pl.ds(b * hist, hist)],
                    out_hbm.at[base + c * bpc + b], ssem[p])

        def wait_store(p):
            for b in range(bpc):
                pltpu.make_async_copy(
                    rows[p].at[pl.ds(0, hist)], out_hbm.at[0], ssem[p]).wait()

        fire_gather(0, 0)
        for c in range(num_chunks):
            p = c % 2
            q = (c + 1) % 2
            if c + 1 < num_chunks:
                if c >= 1:
                    wait_store(q)
                fire_gather(c + 1, q)
            wait_gather(p)
            fire_store(c, p)
        wait_store((num_chunks - 2) % 2)
        wait_store((num_chunks - 1) % 2)

    grid_kernel = pl.kernel(
        body,
        out_type=jax.ShapeDtypeStruct((batches, hist, dim), jnp.float32),
        mesh=plsc.VectorSubcoreMesh(
            core_axis_name="c", subcore_axis_name="s", num_cores=NC,
            num_subcores=NS),
        scratch_types=[
            pltpu.VMEM((num_chunks, 1, CHUNK), jnp.int32),
            pltpu.VMEM((CHUNK, dim), jnp.float32),
            pltpu.VMEM((CHUNK, dim), jnp.float32),
            pltpu.SemaphoreType.DMA,
            pltpu.SemaphoreType.DMA,
            pltpu.SemaphoreType.DMA,
            pltpu.SemaphoreType.DMA,
        ],
        compiler_params=pltpu.CompilerParams(use_tc_tiling_on_sc=False),
    )
    return grid_kernel(idx, table)


def kernel(indices, table):
    batch, hist = indices.shape
    dim = table.shape[1]
    total = batch * hist
    assert total % (NW * CHUNK) == 0 and CHUNK % hist == 0
    num_chunks = total // (NW * CHUNK)
    idx = indices.reshape(NW, num_chunks, 1, CHUNK).astype(jnp.int32)
    return _sc_gather(idx, table, num_chunks=num_chunks, dim=dim, hist=hist)
